# Initial kernel scaffold; baseline (speedup 1.0000x reference)
import jax
import jax.numpy as jnp
from jax.experimental import pallas as pl


def kernel(x, edge_index, expert_vector, bias, alpha, gcn_W, gcn_b, fc_W, fc_b):
    raise NotImplementedError("write your pallas kernel here")


# speedup vs baseline: 7.8063x; 7.8063x over previous
import functools

import jax
import jax.numpy as jnp
from jax.experimental import pallas as pl

N = 10000
D = 128
NUM_EXPERTS = 64
TOP_K = 8
BLOCK_ROWS = 1000


def _router_block(x_ref, w_ref, bias_ref, alpha_ref, out_w_ref, out_i_ref):
    alpha = alpha_ref[0, 0]
    scores = jax.lax.dot_general(
        x_ref[...], w_ref[...],
        dimension_numbers=(((1,), (1,)), ((), ())),
        preferred_element_type=jnp.float32,
    )
    scores = jax.nn.sigmoid(alpha * (scores + bias_ref[...]))

    col = jax.lax.broadcasted_iota(jnp.int32, scores.shape, 1)
    work = scores
    vals = []
    idxs = []
    for _ in range(TOP_K):
        m = jnp.max(work, axis=1, keepdims=True)
        hit = work >= m
        idx = jnp.min(jnp.where(hit, col, NUM_EXPERTS), axis=1, keepdims=True)
        vals.append(m)
        idxs.append(idx)
        work = jnp.where(col == idx, -1.0, work)

    total = vals[0]
    for v in vals[1:]:
        total = total + v
    inv = 1.0 / (total + 1e-6)

    out = jnp.zeros(scores.shape, jnp.float32)
    for v, idx in zip(vals, idxs):
        out = jnp.where(col == idx, v * inv, out)
    out_w_ref[...] = out
    out_i_ref[...] = jnp.concatenate(idxs, axis=1)


def kernel(x, edge_index, expert_vector, bias, alpha, gcn_W, gcn_b, fc_W, fc_b):
    del edge_index, gcn_W, gcn_b, fc_W, fc_b
    n = x.shape[0]
    grid = (n // BLOCK_ROWS,)
    bias2 = bias.reshape(1, NUM_EXPERTS)
    alpha2 = jnp.asarray(alpha, jnp.float32).reshape(1, 1)
    out_w, out_i = pl.pallas_call(
        _router_block,
        grid=grid,
        in_specs=[
            pl.BlockSpec((BLOCK_ROWS, D), lambda i: (i, 0)),
            pl.BlockSpec((NUM_EXPERTS, D), lambda i: (0, 0)),
            pl.BlockSpec((1, NUM_EXPERTS), lambda i: (0, 0)),
            pl.BlockSpec((1, 1), lambda i: (0, 0)),
        ],
        out_specs=[
            pl.BlockSpec((BLOCK_ROWS, NUM_EXPERTS), lambda i: (i, 0)),
            pl.BlockSpec((BLOCK_ROWS, TOP_K), lambda i: (i, 0)),
        ],
        out_shape=[
            jax.ShapeDtypeStruct((n, NUM_EXPERTS), jnp.float32),
            jax.ShapeDtypeStruct((n, TOP_K), jnp.int32),
        ],
    )(x, expert_vector, bias2, alpha2)
    return out_w, out_i
